# baseline (speedup 1.0000x reference)
import functools

import jax
import jax.numpy as jnp
import numpy as np
from jax.experimental import pallas as pl
from jax.experimental.pallas import tpu as pltpu

B, S, H, NL, NH, FF, V, C = 8, 256, 768, 12, 12, 3072, 30522, 9
DH = H // NH
SCALE = float(1.0 / np.sqrt(DH))
R = 32
_VMEM = 52 * 1024 * 1024


def _ln2d(y, w, b):
    m = jnp.mean(y, axis=-1, keepdims=True)
    v = jnp.mean((y - m) ** 2, axis=-1, keepdims=True)
    return (y - m) * jax.lax.rsqrt(v + 1e-12) * w + b


def _embed_kernel(x_sm, word_ref, pos_ref, type_ref, lnw_ref, lnb_ref,
                  out_ref, rows_ref, sem):
    b = pl.program_id(0)
    for s in range(S):
        idx = x_sm[b, s]
        pltpu.make_async_copy(word_ref.at[pl.ds(idx, 1), :],
                              rows_ref.at[pl.ds(s, 1), :], sem).start()
    for s in range(S):
        pltpu.make_async_copy(rows_ref.at[pl.ds(s, 1), :],
                              rows_ref.at[pl.ds(s, 1), :], sem).wait()
    e = rows_ref[...] + pos_ref[...] + type_ref[...]
    out_ref[...] = _ln2d(e, lnw_ref[...], lnb_ref[...]).reshape(1, S, H)


def _embed(x, word_emb, pos_emb, type_emb, lnw, lnb):
    grid_spec = pltpu.PrefetchScalarGridSpec(
        num_scalar_prefetch=1,
        grid=(B,),
        in_specs=[
            pl.BlockSpec(memory_space=pl.ANY),
            pl.BlockSpec((S, H), lambda b, xs: (0, 0)),
            pl.BlockSpec((1, H), lambda b, xs: (0, 0)),
            pl.BlockSpec((1, H), lambda b, xs: (0, 0)),
            pl.BlockSpec((1, H), lambda b, xs: (0, 0)),
        ],
        out_specs=pl.BlockSpec((1, S, H), lambda b, xs: (b, 0, 0)),
        scratch_shapes=[pltpu.VMEM((S, H), jnp.float32),
                        pltpu.SemaphoreType.DMA],
    )
    return pl.pallas_call(
        _embed_kernel,
        grid_spec=grid_spec,
        out_shape=jax.ShapeDtypeStruct((B, S, H), jnp.float32),
        compiler_params=pltpu.CompilerParams(
            dimension_semantics=("arbitrary",),
            vmem_limit_bytes=_VMEM,
        ),
    )(x, word_emb, pos_emb, type_emb, lnw, lnb)


MB = 2
M = MB * S


def _layer_kernel(h_ref, wqkv_ref, bqkv_ref, wo_ref, bo_ref, lnw1_ref,
                  lnb1_ref, w1_ref, b1_ref, w2_ref, b2_ref, lnw2_ref,
                  lnb2_ref, out_ref, wqkvb_ref, wob_ref, w1b_ref, w2b_ref):
    f32 = jnp.float32
    bf = jnp.bfloat16

    @pl.when(pl.program_id(0) == 0)
    def _cast_weights():
        wqkvb_ref[...] = wqkv_ref[...].astype(bf)
        wob_ref[...] = wo_ref[...].astype(bf)
        w1b_ref[...] = w1_ref[...].astype(bf)
        w2b_ref[...] = w2_ref[...].astype(bf)

    h = h_ref[...].reshape(M, H)
    hb = h.astype(bf)
    q = jnp.dot(hb, wqkvb_ref[0], preferred_element_type=f32) + bqkv_ref[0:1, :]
    k = jnp.dot(hb, wqkvb_ref[1], preferred_element_type=f32) + bqkv_ref[1:2, :]
    v = jnp.dot(hb, wqkvb_ref[2], preferred_element_type=f32) + bqkv_ref[2:3, :]
    qb = q.astype(bf)
    kb = k.astype(bf)
    vb = v.astype(bf)
    ctx_rows = []
    for bb in range(MB):
        rs = slice(bb * S, (bb + 1) * S)
        ctx_parts = []
        for hd in range(NH):
            sl = slice(hd * DH, (hd + 1) * DH)
            s = jax.lax.dot_general(qb[rs, sl], kb[rs, sl],
                                    (((1,), (1,)), ((), ())),
                                    preferred_element_type=f32) * SCALE
            m = jnp.max(s, axis=-1, keepdims=True)
            p = jnp.exp(s - m)
            p = p / jnp.sum(p, axis=-1, keepdims=True)
            ctx_parts.append(jax.lax.dot_general(p.astype(bf), vb[rs, sl],
                                                 (((1,), (0,)), ((), ())),
                                                 preferred_element_type=f32))
        ctx_rows.append(jnp.concatenate(ctx_parts, axis=-1))
    ctx = jnp.concatenate(ctx_rows, axis=0)
    attn = jnp.dot(ctx.astype(bf), wob_ref[...],
                   preferred_element_type=f32) + bo_ref[...]
    h1 = _ln2d(h + attn, lnw1_ref[...], lnb1_ref[...])
    f = jnp.dot(h1.astype(bf), w1b_ref[...],
                preferred_element_type=f32) + b1_ref[...]
    f = jax.nn.gelu(f)
    g = jnp.dot(f.astype(bf), w2b_ref[...],
                preferred_element_type=f32) + b2_ref[...]
    out_ref[...] = _ln2d(h1 + g, lnw2_ref[...], lnb2_ref[...]).reshape(1, M, H)


def _layer_call(h, wqkv, bqkv, wo, bo, lnw1, lnb1, w1, b1, w2, b2, lnw2, lnb2):
    n = B // MB
    return pl.pallas_call(
        _layer_kernel,
        grid=(n,),
        in_specs=[
            pl.BlockSpec((1, M, H), lambda b: (b, 0, 0)),
            pl.BlockSpec((3, H, H), lambda b: (0, 0, 0)),
            pl.BlockSpec((3, H), lambda b: (0, 0)),
            pl.BlockSpec((H, H), lambda b: (0, 0)),
            pl.BlockSpec((1, H), lambda b: (0, 0)),
            pl.BlockSpec((1, H), lambda b: (0, 0)),
            pl.BlockSpec((1, H), lambda b: (0, 0)),
            pl.BlockSpec((H, FF), lambda b: (0, 0)),
            pl.BlockSpec((1, FF), lambda b: (0, 0)),
            pl.BlockSpec((FF, H), lambda b: (0, 0)),
            pl.BlockSpec((1, H), lambda b: (0, 0)),
            pl.BlockSpec((1, H), lambda b: (0, 0)),
            pl.BlockSpec((1, H), lambda b: (0, 0)),
        ],
        out_specs=pl.BlockSpec((1, M, H), lambda b: (b, 0, 0)),
        out_shape=jax.ShapeDtypeStruct((n, M, H), jnp.float32),
        scratch_shapes=[pltpu.VMEM((3, H, H), jnp.bfloat16),
                        pltpu.VMEM((H, H), jnp.bfloat16),
                        pltpu.VMEM((H, FF), jnp.bfloat16),
                        pltpu.VMEM((FF, H), jnp.bfloat16)],
        compiler_params=pltpu.CompilerParams(
            dimension_semantics=("arbitrary",),
            vmem_limit_bytes=_VMEM,
        ),
    )(h, wqkv, bqkv, wo, bo, lnw1, lnb1, w1, b1, w2, b2, lnw2, lnb2)


def _crf_kernel(hT_ref, t_ref, clsw_ref, clsb_ref, start_ref, end_ref,
                trans_ref, out_ref, em_ref):
    f32 = jnp.float32
    em = jnp.dot(hT_ref[...].astype(jnp.bfloat16), clsw_ref[...],
                 preferred_element_type=f32) + clsb_ref[...]
    em3 = em.reshape(S, B, C)
    em_ref[...] = em3
    t = t_ref[...]
    iota_c = jax.lax.broadcasted_iota(jnp.int32, (S, B, C), 2)
    onehot = jnp.where(iota_c == t, 1.0, 0.0)
    em_t = jnp.sum(em3 * onehot, axis=2)

    rows = jnp.zeros((S, B, C), f32)
    for kk in range(C):
        rows = rows + jnp.where(t == kk, 1.0, 0.0) * trans_ref[kk:kk + 1, :][None]
    trans_t = jnp.sum(rows[:-1] * onehot[1:], axis=2)

    start_t = jnp.sum(start_ref[...][None] * onehot[0:1], axis=2)
    end_t = jnp.sum(end_ref[...][None] * onehot[S - 1:S], axis=2)
    num = (start_t + em_t[0:1] + end_t
           + jnp.sum(trans_t + em_t[1:], axis=0, keepdims=True))

    alpha0 = start_ref[...][None] + em3[0:1]

    def body(i, alpha):
        em_i = em_ref[pl.ds(i, 1)]
        terms = [alpha[:, :, kk:kk + 1] + trans_ref[kk:kk + 1, :][None]
                 for kk in range(C)]
        m = terms[0]
        for z in terms[1:]:
            m = jnp.maximum(m, z)
        ssum = jnp.exp(terms[0] - m)
        for z in terms[1:]:
            ssum = ssum + jnp.exp(z - m)
        return m + jnp.log(ssum) + em_i

    alpha = jax.lax.fori_loop(1, S, body, alpha0)
    fin = alpha + end_ref[...][None]
    mf = jnp.max(fin, axis=2, keepdims=True)
    den = mf[:, :, 0] + jnp.log(jnp.sum(jnp.exp(fin - mf), axis=2))
    out_ref[...] = jnp.sum(den - num, axis=1, keepdims=True) * (1.0 / B)


def _head_crf(hT, tT, cls_w, cls_b, crf_start, crf_end, crf_trans):
    return pl.pallas_call(
        _crf_kernel,
        grid=(1,),
        in_specs=[
            pl.BlockSpec((S * B, H), lambda i: (0, 0)),
            pl.BlockSpec((S, B, 1), lambda i: (0, 0, 0)),
            pl.BlockSpec((H, C), lambda i: (0, 0)),
            pl.BlockSpec((1, C), lambda i: (0, 0)),
            pl.BlockSpec((1, C), lambda i: (0, 0)),
            pl.BlockSpec((1, C), lambda i: (0, 0)),
            pl.BlockSpec((C, C), lambda i: (0, 0)),
        ],
        out_specs=pl.BlockSpec((1, 1), lambda i: (0, 0)),
        out_shape=jax.ShapeDtypeStruct((1, 1), jnp.float32),
        scratch_shapes=[pltpu.VMEM((S, B, C), jnp.float32)],
        compiler_params=pltpu.CompilerParams(
            dimension_semantics=("arbitrary",),
            vmem_limit_bytes=_VMEM,
        ),
    )(hT, tT, cls_w, cls_b, crf_start, crf_end, crf_trans)


def kernel(x, target, word_emb, pos_emb, type_emb, emb_ln_w, emb_ln_b,
           qkv_w, qkv_b, attn_out_w, attn_out_b, attn_ln_w, attn_ln_b,
           ffn_w1, ffn_b1, ffn_w2, ffn_b2, ffn_ln_w, ffn_ln_b,
           cls_w, cls_b, crf_start, crf_end, crf_trans):
    bf = jnp.bfloat16
    h = _embed(x, word_emb, pos_emb, type_emb,
               emb_ln_w.reshape(1, H), emb_ln_b.reshape(1, H))
    h = h.reshape(B // MB, M, H)
    for l in range(NL):
        h = _layer_call(h, qkv_w[l], qkv_b[l], attn_out_w[l],
                        attn_out_b[l].reshape(1, H),
                        attn_ln_w[l].reshape(1, H), attn_ln_b[l].reshape(1, H),
                        ffn_w1[l], ffn_b1[l].reshape(1, FF), ffn_w2[l],
                        ffn_b2[l].reshape(1, H),
                        ffn_ln_w[l].reshape(1, H), ffn_ln_b[l].reshape(1, H))
    hT = h.reshape(B, S, H).transpose(1, 0, 2).reshape(S * B, H)
    tT = target.T.reshape(S, B, 1)
    loss = _head_crf(hT, tT, cls_w, cls_b.reshape(1, C),
                     crf_start.reshape(1, C), crf_end.reshape(1, C), crf_trans)
    return loss[0, 0]


# speedup vs baseline: 1.0541x; 1.0541x over previous
import functools

import jax
import jax.numpy as jnp
import numpy as np
from jax.experimental import pallas as pl
from jax.experimental.pallas import tpu as pltpu

B, S, H, NL, NH, FF, V, C = 8, 256, 768, 12, 12, 3072, 30522, 9
DH = H // NH
SCALE = float(1.0 / np.sqrt(DH))
R = 32
_VMEM = 52 * 1024 * 1024


def _ln2d(y, w, b):
    m = jnp.mean(y, axis=-1, keepdims=True)
    v = jnp.mean((y - m) ** 2, axis=-1, keepdims=True)
    return (y - m) * jax.lax.rsqrt(v + 1e-12) * w + b


def _embed_kernel(x_sm, word_ref, pos_ref, type_ref, lnw_ref, lnb_ref,
                  out_ref, rows_ref, sem):
    b = pl.program_id(0)
    for s in range(S):
        idx = x_sm[b, s]
        pltpu.make_async_copy(word_ref.at[pl.ds(idx, 1), :],
                              rows_ref.at[pl.ds(s, 1), :], sem).start()
    for s in range(S):
        pltpu.make_async_copy(rows_ref.at[pl.ds(s, 1), :],
                              rows_ref.at[pl.ds(s, 1), :], sem).wait()
    e = rows_ref[...] + pos_ref[...] + type_ref[...]
    out_ref[...] = _ln2d(e, lnw_ref[...], lnb_ref[...]).reshape(1, S, H)


def _embed(x, word_emb, pos_emb, type_emb, lnw, lnb):
    grid_spec = pltpu.PrefetchScalarGridSpec(
        num_scalar_prefetch=1,
        grid=(B,),
        in_specs=[
            pl.BlockSpec(memory_space=pl.ANY),
            pl.BlockSpec((S, H), lambda b, xs: (0, 0)),
            pl.BlockSpec((1, H), lambda b, xs: (0, 0)),
            pl.BlockSpec((1, H), lambda b, xs: (0, 0)),
            pl.BlockSpec((1, H), lambda b, xs: (0, 0)),
        ],
        out_specs=pl.BlockSpec((1, S, H), lambda b, xs: (b, 0, 0)),
        scratch_shapes=[pltpu.VMEM((S, H), jnp.float32),
                        pltpu.SemaphoreType.DMA],
    )
    return pl.pallas_call(
        _embed_kernel,
        grid_spec=grid_spec,
        out_shape=jax.ShapeDtypeStruct((B, S, H), jnp.float32),
        compiler_params=pltpu.CompilerParams(
            dimension_semantics=("arbitrary",),
            vmem_limit_bytes=_VMEM,
        ),
    )(x, word_emb, pos_emb, type_emb, lnw, lnb)


MB = 2
M = MB * S


def _layer_kernel(h_ref, wqkv_ref, bqkv_ref, wo_ref, bo_ref, lnw1_ref,
                  lnb1_ref, w1_ref, b1_ref, w2_ref, b2_ref, lnw2_ref,
                  lnb2_ref, out_ref):
    f32 = jnp.float32
    bf = jnp.bfloat16
    h = h_ref[...].reshape(M, H)
    hb = h.astype(bf)
    q = jnp.dot(hb, wqkv_ref[0], preferred_element_type=f32) + bqkv_ref[0:1, :]
    k = jnp.dot(hb, wqkv_ref[1], preferred_element_type=f32) + bqkv_ref[1:2, :]
    v = jnp.dot(hb, wqkv_ref[2], preferred_element_type=f32) + bqkv_ref[2:3, :]
    qb = q.astype(bf)
    kb = k.astype(bf)
    vb = v.astype(bf)
    ctx_rows = []
    for bb in range(MB):
        rs = slice(bb * S, (bb + 1) * S)
        ctx_parts = []
        for hd in range(NH):
            sl = slice(hd * DH, (hd + 1) * DH)
            s = jax.lax.dot_general(qb[rs, sl], kb[rs, sl],
                                    (((1,), (1,)), ((), ())),
                                    preferred_element_type=f32) * SCALE
            m = jnp.max(s, axis=-1, keepdims=True)
            p = jnp.exp(s - m)
            p = p / jnp.sum(p, axis=-1, keepdims=True)
            ctx_parts.append(jax.lax.dot_general(p.astype(bf), vb[rs, sl],
                                                 (((1,), (0,)), ((), ())),
                                                 preferred_element_type=f32))
        ctx_rows.append(jnp.concatenate(ctx_parts, axis=-1))
    ctx = jnp.concatenate(ctx_rows, axis=0)
    attn = jnp.dot(ctx.astype(bf), wo_ref[...],
                   preferred_element_type=f32) + bo_ref[...]
    h1 = _ln2d(h + attn, lnw1_ref[...], lnb1_ref[...])
    f = (jnp.dot(h1.astype(bf), w1_ref[...],
                 preferred_element_type=f32) + b1_ref[...]).astype(bf)
    f = jax.nn.gelu(f)
    g = jnp.dot(f, w2_ref[...], preferred_element_type=f32) + b2_ref[...]
    out_ref[...] = _ln2d(h1 + g, lnw2_ref[...], lnb2_ref[...]).reshape(1, M, H)


def _layer_call(h, wqkv, bqkv, wo, bo, lnw1, lnb1, w1, b1, w2, b2, lnw2, lnb2):
    n = B // MB
    return pl.pallas_call(
        _layer_kernel,
        grid=(n,),
        in_specs=[
            pl.BlockSpec((1, M, H), lambda b: (b, 0, 0)),
            pl.BlockSpec((3, H, H), lambda b: (0, 0, 0)),
            pl.BlockSpec((3, H), lambda b: (0, 0)),
            pl.BlockSpec((H, H), lambda b: (0, 0)),
            pl.BlockSpec((1, H), lambda b: (0, 0)),
            pl.BlockSpec((1, H), lambda b: (0, 0)),
            pl.BlockSpec((1, H), lambda b: (0, 0)),
            pl.BlockSpec((H, FF), lambda b: (0, 0)),
            pl.BlockSpec((1, FF), lambda b: (0, 0)),
            pl.BlockSpec((FF, H), lambda b: (0, 0)),
            pl.BlockSpec((1, H), lambda b: (0, 0)),
            pl.BlockSpec((1, H), lambda b: (0, 0)),
            pl.BlockSpec((1, H), lambda b: (0, 0)),
        ],
        out_specs=pl.BlockSpec((1, M, H), lambda b: (b, 0, 0)),
        out_shape=jax.ShapeDtypeStruct((n, M, H), jnp.float32),
        compiler_params=pltpu.CompilerParams(
            dimension_semantics=("arbitrary",),
            vmem_limit_bytes=_VMEM,
        ),
    )(h, wqkv, bqkv, wo, bo, lnw1, lnb1, w1, b1, w2, b2, lnw2, lnb2)


def _crf_kernel(hT_ref, t_ref, clsw_ref, clsb_ref, start_ref, end_ref,
                trans_ref, out_ref, em_ref):
    f32 = jnp.float32
    em = jnp.dot(hT_ref[...].astype(jnp.bfloat16), clsw_ref[...],
                 preferred_element_type=f32) + clsb_ref[...]
    em3 = em.reshape(S, B, C)
    em_ref[...] = em3
    t = t_ref[...]
    iota_c = jax.lax.broadcasted_iota(jnp.int32, (S, B, C), 2)
    onehot = jnp.where(iota_c == t, 1.0, 0.0)
    em_t = jnp.sum(em3 * onehot, axis=2)

    rows = jnp.zeros((S, B, C), f32)
    for kk in range(C):
        rows = rows + jnp.where(t == kk, 1.0, 0.0) * trans_ref[kk:kk + 1, :][None]
    trans_t = jnp.sum(rows[:-1] * onehot[1:], axis=2)

    start_t = jnp.sum(start_ref[...][None] * onehot[0:1], axis=2)
    end_t = jnp.sum(end_ref[...][None] * onehot[S - 1:S], axis=2)
    num = (start_t + em_t[0:1] + end_t
           + jnp.sum(trans_t + em_t[1:], axis=0, keepdims=True))

    alpha0 = start_ref[...][None] + em3[0:1]

    def body(i, alpha):
        em_i = em_ref[pl.ds(i, 1)]
        terms = [alpha[:, :, kk:kk + 1] + trans_ref[kk:kk + 1, :][None]
                 for kk in range(C)]
        m = terms[0]
        for z in terms[1:]:
            m = jnp.maximum(m, z)
        ssum = jnp.exp(terms[0] - m)
        for z in terms[1:]:
            ssum = ssum + jnp.exp(z - m)
        return m + jnp.log(ssum) + em_i

    alpha = jax.lax.fori_loop(1, S, body, alpha0)
    fin = alpha + end_ref[...][None]
    mf = jnp.max(fin, axis=2, keepdims=True)
    den = mf[:, :, 0] + jnp.log(jnp.sum(jnp.exp(fin - mf), axis=2))
    out_ref[...] = jnp.sum(den - num, axis=1, keepdims=True) * (1.0 / B)


def _head_crf(hT, tT, cls_w, cls_b, crf_start, crf_end, crf_trans):
    return pl.pallas_call(
        _crf_kernel,
        grid=(1,),
        in_specs=[
            pl.BlockSpec((S * B, H), lambda i: (0, 0)),
            pl.BlockSpec((S, B, 1), lambda i: (0, 0, 0)),
            pl.BlockSpec((H, C), lambda i: (0, 0)),
            pl.BlockSpec((1, C), lambda i: (0, 0)),
            pl.BlockSpec((1, C), lambda i: (0, 0)),
            pl.BlockSpec((1, C), lambda i: (0, 0)),
            pl.BlockSpec((C, C), lambda i: (0, 0)),
        ],
        out_specs=pl.BlockSpec((1, 1), lambda i: (0, 0)),
        out_shape=jax.ShapeDtypeStruct((1, 1), jnp.float32),
        scratch_shapes=[pltpu.VMEM((S, B, C), jnp.float32)],
        compiler_params=pltpu.CompilerParams(
            dimension_semantics=("arbitrary",),
            vmem_limit_bytes=_VMEM,
        ),
    )(hT, tT, cls_w, cls_b, crf_start, crf_end, crf_trans)


def kernel(x, target, word_emb, pos_emb, type_emb, emb_ln_w, emb_ln_b,
           qkv_w, qkv_b, attn_out_w, attn_out_b, attn_ln_w, attn_ln_b,
           ffn_w1, ffn_b1, ffn_w2, ffn_b2, ffn_ln_w, ffn_ln_b,
           cls_w, cls_b, crf_start, crf_end, crf_trans):
    bf = jnp.bfloat16
    h = _embed(x, word_emb, pos_emb, type_emb,
               emb_ln_w.reshape(1, H), emb_ln_b.reshape(1, H))
    qkv_wb = qkv_w.astype(bf)
    wo_b = attn_out_w.astype(bf)
    w1_b = ffn_w1.astype(bf)
    w2_b = ffn_w2.astype(bf)
    h = h.reshape(B // MB, M, H)
    for l in range(NL):
        h = _layer_call(h, qkv_wb[l], qkv_b[l], wo_b[l],
                        attn_out_b[l].reshape(1, H),
                        attn_ln_w[l].reshape(1, H), attn_ln_b[l].reshape(1, H),
                        w1_b[l], ffn_b1[l].reshape(1, FF), w2_b[l],
                        ffn_b2[l].reshape(1, H),
                        ffn_ln_w[l].reshape(1, H), ffn_ln_b[l].reshape(1, H))
    hT = h.reshape(B, S, H).transpose(1, 0, 2).reshape(S * B, H)
    tT = target.T.reshape(S, B, 1)
    loss = _head_crf(hT, tT, cls_w, cls_b.reshape(1, C),
                     crf_start.reshape(1, C), crf_end.reshape(1, C), crf_trans)
    return loss[0, 0]
